# Initial kernel scaffold; baseline (speedup 1.0000x reference)
import functools

import jax
import jax.numpy as jnp
from jax import lax
from jax.experimental import pallas as pl
from jax.experimental.pallas import tpu as pltpu
from jax.experimental.pallas import tpu_sc as plsc

N = 10000
E = 320000
D_IN = 128
H = 240
S0 = 64
L = 3
NG = 64
LATENT = 128

HP = 256
NP_ = 10240
EP = 327680
CHUNK = 128
NCORE = 2
NSUB = 16
NW = NCORE * NSUB


def _make_sc_gather(width, kb):
    mesh = plsc.VectorSubcoreMesh(core_axis_name="c", subcore_axis_name="s")
    cpw = (EP // NW) // CHUNK

    @functools.partial(
        pl.kernel, mesh=mesh,
        out_type=jax.ShapeDtypeStruct((EP, width), jnp.float32),
        scratch_types=[
            pltpu.VMEM((kb, CHUNK), jnp.int32),
            pltpu.VMEM((kb * CHUNK, width), jnp.float32),
            pltpu.SemaphoreType.DMA,
        ],
    )
    def k(table, idx, out, idx_v, rows_v, sem):
        wid = lax.axis_index("s") * NCORE + lax.axis_index("c")
        cbase = wid * cpw

        @pl.loop(0, cpw, step=kb)
        def _(i):
            pltpu.sync_copy(idx.at[pl.ds(cbase + i, kb)], idx_v)
            descs = [
                pltpu.async_copy(table.at[idx_v.at[j]],
                                 rows_v.at[pl.ds(j * CHUNK, CHUNK)], sem)
                for j in range(kb)
            ]
            for d in descs:
                d.wait()
            pltpu.sync_copy(rows_v, out.at[pl.ds((cbase + i) * CHUNK, kb * CHUNK)])

    return k


def _make_sc_scatter_add(kb):
    mesh = plsc.VectorSubcoreMesh(core_axis_name="c", subcore_axis_name="s")
    eps = EP // NSUB
    cps = eps // CHUNK
    rps = NP_ // NSUB

    @functools.partial(
        pl.kernel, mesh=mesh,
        out_type=jax.ShapeDtypeStruct((NP_, HP), jnp.float32),
        scratch_types=[
            pltpu.VMEM((kb, CHUNK), jnp.int32),
            pltpu.VMEM((kb * CHUNK, 128), jnp.float32),
            pltpu.VMEM_SHARED((NP_, 128), jnp.float32),
            pltpu.SemaphoreType.DMA,
        ],
    )
    def k(gated, idx, zeros, agg, idx_v, dat_v, acc_sh, sem):
        c = lax.axis_index("c")
        s = lax.axis_index("s")
        pltpu.sync_copy(zeros.at[pl.ds(s * rps, rps)],
                        acc_sh.at[pl.ds(s * rps, rps)])
        plsc.subcore_barrier()
        cbase = s * cps

        @pl.loop(0, cps, step=kb)
        def _(i):
            pltpu.sync_copy(idx.at[pl.ds(cbase + i, kb)], idx_v)
            pltpu.async_copy(
                gated.at[pl.ds((cbase + i) * CHUNK, kb * CHUNK),
                         pl.ds(c * 128, 128)],
                dat_v, sem).wait()
            for j in range(kb):
                pltpu.sync_copy(dat_v.at[pl.ds(j * CHUNK, CHUNK)],
                                acc_sh.at[idx_v.at[j]], add=True)

        plsc.subcore_barrier()
        pltpu.sync_copy(acc_sh.at[pl.ds(s * rps, rps)],
                        agg.at[pl.ds(s * rps, rps), pl.ds(c * 128, 128)])

    return k


_sc_gather16 = _make_sc_gather(16, 8)
_sc_gather256 = _make_sc_gather(HP, 2)
_sc_scatter = _make_sc_scatter_add(4)


_BLK_N = 1024
_BLK_E = 1024


def _edge_attr_call(pr, pc):
    def body(pr_ref, pc_ref, out_ref):
        r = pr_ref[...] - pc_ref[...]
        d2 = jnp.sum(r * r, axis=1, keepdims=True)
        dist = jnp.sqrt(d2)
        inv = 1.0 / (dist + 1e-8)
        lane = lax.broadcasted_iota(jnp.int32, (_BLK_E, 16), 1)
        out_ref[...] = r * inv + jnp.where(lane == 3, dist, 0.0)

    return pl.pallas_call(
        body,
        grid=(EP // _BLK_E,),
        in_specs=[pl.BlockSpec((_BLK_E, 16), lambda i: (i, 0)),
                  pl.BlockSpec((_BLK_E, 16), lambda i: (i, 0))],
        out_specs=pl.BlockSpec((_BLK_E, 16), lambda i: (i, 0)),
        out_shape=jax.ShapeDtypeStruct((EP, 16), jnp.float32),
    )(pr, pc)


def _prologue_call(xp, w_in, wn0p):
    def body(x_ref, win_ref, wn_ref, h_ref, g_ref):
        h = jnp.dot(x_ref[...], win_ref[...], preferred_element_type=jnp.float32)
        h_ref[...] = h
        g_ref[...] = jnp.dot(h, wn_ref[...], preferred_element_type=jnp.float32)

    return pl.pallas_call(
        body,
        grid=(NP_ // _BLK_N,),
        in_specs=[pl.BlockSpec((_BLK_N, D_IN), lambda i: (i, 0)),
                  pl.BlockSpec((D_IN, H), lambda i: (0, 0)),
                  pl.BlockSpec((H, HP), lambda i: (0, 0))],
        out_specs=[pl.BlockSpec((_BLK_N, H), lambda i: (i, 0)),
                   pl.BlockSpec((_BLK_N, HP), lambda i: (i, 0))],
        out_shape=[jax.ShapeDtypeStruct((NP_, H), jnp.float32),
                   jax.ShapeDtypeStruct((NP_, HP), jnp.float32)],
    )(xp, w_in, wn0p)


def _gate_call(gath, ea, wep, wgext):
    def body(g_ref, ea_ref, we_ref, wg_ref, out_ref):
        a = jnp.dot(ea_ref[...], we_ref[...], preferred_element_type=jnp.float32)
        m = g_ref[...] * a
        z = jnp.dot(m[:, :S0], wg_ref[...], preferred_element_type=jnp.float32)
        out_ref[...] = m * jax.nn.sigmoid(z)

    return pl.pallas_call(
        body,
        grid=(EP // _BLK_E,),
        in_specs=[pl.BlockSpec((_BLK_E, HP), lambda i: (i, 0)),
                  pl.BlockSpec((_BLK_E, 16), lambda i: (i, 0)),
                  pl.BlockSpec((16, HP), lambda i: (0, 0)),
                  pl.BlockSpec((S0, HP), lambda i: (0, 0))],
        out_specs=pl.BlockSpec((_BLK_E, HP), lambda i: (i, 0)),
        out_shape=jax.ShapeDtypeStruct((EP, HP), jnp.float32),
    )(gath, ea, wep, wgext)


def _update_call(h, agg, wlin_msg, wn_upd, wm_upd, wg_ext, wlin_upd, wn_next):
    has_next = wn_next is not None

    def body(h_ref, agg_ref, wl_ref, wn_ref, wm_ref, wg_ref, wu_ref,
             *rest):
        if has_next:
            wnx_ref, h_out, g_out = rest
        else:
            (h_out,) = rest
        h_ = h_ref[...]
        aggw = jnp.dot(agg_ref[:, :H], wl_ref[...],
                       preferred_element_type=jnp.float32)
        u1 = (jnp.dot(h_, wn_ref[...], preferred_element_type=jnp.float32)
              * jnp.dot(aggw, wm_ref[...], preferred_element_type=jnp.float32))
        z = jnp.dot(u1[:, :S0], wg_ref[...], preferred_element_type=jnp.float32)
        u2 = u1 * jax.nn.sigmoid(z)
        hn = h_ + jnp.dot(u2, wu_ref[...], preferred_element_type=jnp.float32)
        h_out[...] = hn
        if has_next:
            g_out[...] = jnp.dot(hn, wnx_ref[...],
                                 preferred_element_type=jnp.float32)

    in_specs = [pl.BlockSpec((_BLK_N, H), lambda i: (i, 0)),
                pl.BlockSpec((_BLK_N, HP), lambda i: (i, 0)),
                pl.BlockSpec((H, H), lambda i: (0, 0)),
                pl.BlockSpec((H, H), lambda i: (0, 0)),
                pl.BlockSpec((H, H), lambda i: (0, 0)),
                pl.BlockSpec((S0, H), lambda i: (0, 0)),
                pl.BlockSpec((H, H), lambda i: (0, 0))]
    out_specs = [pl.BlockSpec((_BLK_N, H), lambda i: (i, 0))]
    out_shape = [jax.ShapeDtypeStruct((NP_, H), jnp.float32)]
    args = [h, agg, wlin_msg, wn_upd, wm_upd, wg_ext, wlin_upd]
    if has_next:
        in_specs.append(pl.BlockSpec((H, HP), lambda i: (0, 0)))
        out_specs.append(pl.BlockSpec((_BLK_N, HP), lambda i: (i, 0)))
        out_shape.append(jax.ShapeDtypeStruct((NP_, HP), jnp.float32))
        args.append(wn_next)

    return pl.pallas_call(
        body,
        grid=(NP_ // _BLK_N,),
        in_specs=in_specs,
        out_specs=out_specs,
        out_shape=out_shape,
    )(*args)


def _pool_call(h, batch2, w_final):
    nb = NP_ // _BLK_N

    def body(h_ref, b_ref, wf_ref, out_ref, acc_s, cnt_s):
        i = pl.program_id(0)

        @pl.when(i == 0)
        def _():
            acc_s[...] = jnp.zeros_like(acc_s)
            cnt_s[...] = jnp.zeros_like(cnt_s)

        inv = h_ref[:, :S0]
        b = b_ref[...]
        oh = (b == lax.broadcasted_iota(jnp.int32, (_BLK_N, NG), 1)
              ).astype(jnp.float32)
        acc_s[...] += lax.dot_general(oh, inv, (((0,), (0,)), ((), ())),
                                      preferred_element_type=jnp.float32)
        cnt_s[...] += lax.dot_general(oh, jnp.ones((_BLK_N, 8), jnp.float32),
                                      (((0,), (0,)), ((), ())),
                                      preferred_element_type=jnp.float32)

        @pl.when(i == nb - 1)
        def _():
            pooled = acc_s[...] / jnp.maximum(cnt_s[:, :1], 1.0)
            out_ref[...] = jnp.dot(pooled, wf_ref[...],
                                   preferred_element_type=jnp.float32)

    return pl.pallas_call(
        body,
        grid=(nb,),
        in_specs=[pl.BlockSpec((_BLK_N, H), lambda i: (i, 0)),
                  pl.BlockSpec((_BLK_N, 1), lambda i: (i, 0)),
                  pl.BlockSpec((S0, LATENT), lambda i: (0, 0))],
        out_specs=pl.BlockSpec((NG, LATENT), lambda i: (0, 0)),
        out_shape=jax.ShapeDtypeStruct((NG, LATENT), jnp.float32),
        scratch_shapes=[pltpu.VMEM((NG, S0), jnp.float32),
                        pltpu.VMEM((NG, 8), jnp.float32)],
    )(h, batch2, w_final)


@jax.jit
def kernel(x, pos, edge_index, batch, W_in, Wn_msg, We_msg, Wg_msg, Wlin_msg,
           Wn_upd, Wm_upd, Wg_upd, Wlin_upd, W_final):
    f32 = jnp.float32
    row = edge_index[0]
    col = edge_index[1]

    xp = jnp.zeros((NP_, D_IN), f32).at[:N].set(x)
    pos16 = jnp.zeros((NP_, 16), f32).at[:N, :3].set(pos)
    rowp = jnp.concatenate(
        [row, jnp.full((EP - E,), N, jnp.int32)]).reshape(EP // CHUNK, CHUNK)
    colp = jnp.concatenate(
        [col, jnp.zeros((EP - E,), jnp.int32)]).reshape(EP // CHUNK, CHUNK)
    batch2 = jnp.concatenate(
        [batch, jnp.full((NP_ - N,), NG, jnp.int32)]).reshape(NP_, 1)
    zeros_acc = jnp.zeros((NP_, 128), f32)

    eye = jnp.eye(S0, dtype=f32)
    wn_msg_p = jnp.zeros((L, H, HP), f32).at[:, :, :H].set(Wn_msg)
    we_p = jnp.zeros((L, 16, HP), f32).at[:, :4, :H].set(We_msg)
    wg_msg_ext = jnp.zeros((L, S0, HP), f32)
    wg_msg_ext = wg_msg_ext.at[:, :, :S0].set(eye)
    wg_msg_ext = wg_msg_ext.at[:, :, S0:H].set(Wg_msg)
    wg_upd_ext = jnp.concatenate(
        [jnp.broadcast_to(eye, (L, S0, S0)), Wg_upd], axis=2)

    pr = _sc_gather16(pos16, rowp)
    pc = _sc_gather16(pos16, colp)
    ea = _edge_attr_call(pr, pc)

    h, g = _prologue_call(xp, W_in, wn_msg_p[0])
    for l in range(L):
        gath = _sc_gather256(g, colp)
        gated = _gate_call(gath, ea, we_p[l], wg_msg_ext[l])
        agg = _sc_scatter(gated, rowp, zeros_acc)
        wn_next = wn_msg_p[l + 1] if l + 1 < L else None
        res = _update_call(h, agg, Wlin_msg[l], Wn_upd[l], Wm_upd[l],
                           wg_upd_ext[l], Wlin_upd[l], wn_next)
        if wn_next is not None:
            h, g = res
        else:
            (h,) = res

    return _pool_call(h, batch2, W_final)


# speedup vs baseline: 1.3959x; 1.3959x over previous
import functools

import jax
import jax.numpy as jnp
from jax import lax
from jax.experimental import pallas as pl
from jax.experimental.pallas import tpu as pltpu
from jax.experimental.pallas import tpu_sc as plsc

N = 10000
E = 320000
D_IN = 128
H = 240
S0 = 64
L = 3
NG = 64
LATENT = 128

HP = 256
NP_ = 10240
EP = 327680
CHUNK = 128
NCORE = 2
NSUB = 16
NW = NCORE * NSUB


def _make_sc_gather(width, kb):
    mesh = plsc.VectorSubcoreMesh(core_axis_name="c", subcore_axis_name="s")
    cpw = (EP // NW) // CHUNK

    @functools.partial(
        pl.kernel, mesh=mesh,
        out_type=jax.ShapeDtypeStruct((EP, width), jnp.float32),
        compiler_params=pltpu.CompilerParams(use_tc_tiling_on_sc=False),
        scratch_types=[
            pltpu.VMEM((kb, CHUNK), jnp.int32),
            pltpu.VMEM((kb * CHUNK, width), jnp.float32),
            pltpu.SemaphoreType.DMA,
        ],
    )
    def k(table, idx, out, idx_v, rows_v, sem):
        wid = lax.axis_index("s") * NCORE + lax.axis_index("c")
        cbase = wid * cpw

        @pl.loop(0, cpw, step=kb)
        def _(i):
            pltpu.sync_copy(idx.at[pl.ds(cbase + i, kb)], idx_v)
            descs = [
                pltpu.async_copy(table.at[idx_v.at[j]],
                                 rows_v.at[pl.ds(j * CHUNK, CHUNK)], sem)
                for j in range(kb)
            ]
            for d in descs:
                d.wait()
            pltpu.sync_copy(rows_v, out.at[pl.ds((cbase + i) * CHUNK, kb * CHUNK)])

    return k


def _make_sc_scatter_add(kb):
    mesh = plsc.VectorSubcoreMesh(core_axis_name="c", subcore_axis_name="s")
    eps = EP // NSUB
    cps = eps // CHUNK
    rps = NP_ // NSUB

    @functools.partial(
        pl.kernel, mesh=mesh,
        out_type=jax.ShapeDtypeStruct((NP_, HP), jnp.float32),
        compiler_params=pltpu.CompilerParams(use_tc_tiling_on_sc=False),
        scratch_types=[
            pltpu.VMEM((kb, CHUNK), jnp.int32),
            pltpu.VMEM((kb * CHUNK, 128), jnp.float32),
            pltpu.VMEM_SHARED((NP_, 128), jnp.float32),
            pltpu.SemaphoreType.DMA,
        ],
    )
    def k(gated, idx, zeros, agg, idx_v, dat_v, acc_sh, sem):
        c = lax.axis_index("c")
        s = lax.axis_index("s")
        pltpu.sync_copy(zeros.at[pl.ds(s * rps, rps)],
                        acc_sh.at[pl.ds(s * rps, rps)])
        plsc.subcore_barrier()
        cbase = s * cps

        @pl.loop(0, cps, step=kb)
        def _(i):
            pltpu.sync_copy(idx.at[pl.ds(cbase + i, kb)], idx_v)
            pltpu.async_copy(
                gated.at[pl.ds((cbase + i) * CHUNK, kb * CHUNK),
                         pl.ds(c * 128, 128)],
                dat_v, sem).wait()
            for j in range(kb):
                pltpu.sync_copy(dat_v.at[pl.ds(j * CHUNK, CHUNK)],
                                acc_sh.at[idx_v.at[j]], add=True)

        plsc.subcore_barrier()
        pltpu.sync_copy(acc_sh.at[pl.ds(s * rps, rps)],
                        agg.at[pl.ds(s * rps, rps), pl.ds(c * 128, 128)])

    return k


_sc_gather_cached = functools.lru_cache(None)(_make_sc_gather)
_sc_scatter_cached = functools.lru_cache(None)(_make_sc_scatter_add)


def _sc_gather16(table, idx):
    return _sc_gather_cached(16, 8)(table, idx)


def _sc_gather256(table, idx):
    return _sc_gather_cached(HP, 2)(table, idx)


def _sc_scatter(gated, idx, zeros):
    return _sc_scatter_cached(1)(gated, idx, zeros)


_BLK_N = 1024
_BLK_E = 1024


def _edge_attr_call(pr, pc):
    def body(pr_ref, pc_ref, out_ref):
        r = pr_ref[...] - pc_ref[...]
        d2 = jnp.sum(r * r, axis=1, keepdims=True)
        dist = jnp.sqrt(d2)
        inv = 1.0 / (dist + 1e-8)
        lane = lax.broadcasted_iota(jnp.int32, (_BLK_E, 16), 1)
        out_ref[...] = r * inv + jnp.where(lane == 3, dist, 0.0)

    return pl.pallas_call(
        body,
        grid=(EP // _BLK_E,),
        in_specs=[pl.BlockSpec((_BLK_E, 16), lambda i: (i, 0)),
                  pl.BlockSpec((_BLK_E, 16), lambda i: (i, 0))],
        out_specs=pl.BlockSpec((_BLK_E, 16), lambda i: (i, 0)),
        out_shape=jax.ShapeDtypeStruct((EP, 16), jnp.float32),
    )(pr, pc)


def _prologue_call(xp, w_in, wn0p):
    def body(x_ref, win_ref, wn_ref, h_ref, g_ref):
        h = jnp.dot(x_ref[...], win_ref[...], preferred_element_type=jnp.float32)
        h_ref[...] = h
        g_ref[...] = jnp.dot(h, wn_ref[...], preferred_element_type=jnp.float32)

    return pl.pallas_call(
        body,
        grid=(NP_ // _BLK_N,),
        in_specs=[pl.BlockSpec((_BLK_N, D_IN), lambda i: (i, 0)),
                  pl.BlockSpec((D_IN, H), lambda i: (0, 0)),
                  pl.BlockSpec((H, HP), lambda i: (0, 0))],
        out_specs=[pl.BlockSpec((_BLK_N, H), lambda i: (i, 0)),
                   pl.BlockSpec((_BLK_N, HP), lambda i: (i, 0))],
        out_shape=[jax.ShapeDtypeStruct((NP_, H), jnp.float32),
                   jax.ShapeDtypeStruct((NP_, HP), jnp.float32)],
    )(xp, w_in, wn0p)


def _gate_call(gath, ea, wep, wgext):
    def body(g_ref, ea_ref, we_ref, wg_ref, out_ref):
        a = jnp.dot(ea_ref[...], we_ref[...], preferred_element_type=jnp.float32)
        m = g_ref[...] * a
        z = jnp.dot(m[:, :S0], wg_ref[...], preferred_element_type=jnp.float32)
        out_ref[...] = m * jax.nn.sigmoid(z)

    return pl.pallas_call(
        body,
        grid=(EP // _BLK_E,),
        in_specs=[pl.BlockSpec((_BLK_E, HP), lambda i: (i, 0)),
                  pl.BlockSpec((_BLK_E, 16), lambda i: (i, 0)),
                  pl.BlockSpec((16, HP), lambda i: (0, 0)),
                  pl.BlockSpec((S0, HP), lambda i: (0, 0))],
        out_specs=pl.BlockSpec((_BLK_E, HP), lambda i: (i, 0)),
        out_shape=jax.ShapeDtypeStruct((EP, HP), jnp.float32),
    )(gath, ea, wep, wgext)


def _update_call(h, agg, wlin_msg, wn_upd, wm_upd, wg_ext, wlin_upd, wn_next):
    has_next = wn_next is not None

    def body(h_ref, agg_ref, wl_ref, wn_ref, wm_ref, wg_ref, wu_ref,
             *rest):
        if has_next:
            wnx_ref, h_out, g_out = rest
        else:
            (h_out,) = rest
        h_ = h_ref[...]
        aggw = jnp.dot(agg_ref[:, :H], wl_ref[...],
                       preferred_element_type=jnp.float32)
        u1 = (jnp.dot(h_, wn_ref[...], preferred_element_type=jnp.float32)
              * jnp.dot(aggw, wm_ref[...], preferred_element_type=jnp.float32))
        z = jnp.dot(u1[:, :S0], wg_ref[...], preferred_element_type=jnp.float32)
        u2 = u1 * jax.nn.sigmoid(z)
        hn = h_ + jnp.dot(u2, wu_ref[...], preferred_element_type=jnp.float32)
        h_out[...] = hn
        if has_next:
            g_out[...] = jnp.dot(hn, wnx_ref[...],
                                 preferred_element_type=jnp.float32)

    in_specs = [pl.BlockSpec((_BLK_N, H), lambda i: (i, 0)),
                pl.BlockSpec((_BLK_N, HP), lambda i: (i, 0)),
                pl.BlockSpec((H, H), lambda i: (0, 0)),
                pl.BlockSpec((H, H), lambda i: (0, 0)),
                pl.BlockSpec((H, H), lambda i: (0, 0)),
                pl.BlockSpec((S0, H), lambda i: (0, 0)),
                pl.BlockSpec((H, H), lambda i: (0, 0))]
    out_specs = [pl.BlockSpec((_BLK_N, H), lambda i: (i, 0))]
    out_shape = [jax.ShapeDtypeStruct((NP_, H), jnp.float32)]
    args = [h, agg, wlin_msg, wn_upd, wm_upd, wg_ext, wlin_upd]
    if has_next:
        in_specs.append(pl.BlockSpec((H, HP), lambda i: (0, 0)))
        out_specs.append(pl.BlockSpec((_BLK_N, HP), lambda i: (i, 0)))
        out_shape.append(jax.ShapeDtypeStruct((NP_, HP), jnp.float32))
        args.append(wn_next)

    return pl.pallas_call(
        body,
        grid=(NP_ // _BLK_N,),
        in_specs=in_specs,
        out_specs=out_specs,
        out_shape=out_shape,
    )(*args)


def _pool_call(h, batch2, w_final):
    nb = NP_ // _BLK_N

    def body(h_ref, b_ref, wf_ref, out_ref, acc_s, cnt_s):
        i = pl.program_id(0)

        @pl.when(i == 0)
        def _():
            acc_s[...] = jnp.zeros_like(acc_s)
            cnt_s[...] = jnp.zeros_like(cnt_s)

        inv = h_ref[:, :S0]
        b = b_ref[...]
        oh = (b == lax.broadcasted_iota(jnp.int32, (_BLK_N, NG), 1)
              ).astype(jnp.float32)
        acc_s[...] += lax.dot_general(oh, inv, (((0,), (0,)), ((), ())),
                                      preferred_element_type=jnp.float32)
        cnt_s[...] += lax.dot_general(oh, jnp.ones((_BLK_N, 8), jnp.float32),
                                      (((0,), (0,)), ((), ())),
                                      preferred_element_type=jnp.float32)

        @pl.when(i == nb - 1)
        def _():
            pooled = acc_s[...] / jnp.maximum(cnt_s[:, :1], 1.0)
            out_ref[...] = jnp.dot(pooled, wf_ref[...],
                                   preferred_element_type=jnp.float32)

    return pl.pallas_call(
        body,
        grid=(nb,),
        in_specs=[pl.BlockSpec((_BLK_N, H), lambda i: (i, 0)),
                  pl.BlockSpec((_BLK_N, 1), lambda i: (i, 0)),
                  pl.BlockSpec((S0, LATENT), lambda i: (0, 0))],
        out_specs=pl.BlockSpec((NG, LATENT), lambda i: (0, 0)),
        out_shape=jax.ShapeDtypeStruct((NG, LATENT), jnp.float32),
        scratch_shapes=[pltpu.VMEM((NG, S0), jnp.float32),
                        pltpu.VMEM((NG, 8), jnp.float32)],
    )(h, batch2, w_final)


@jax.jit
def kernel(x, pos, edge_index, batch, W_in, Wn_msg, We_msg, Wg_msg, Wlin_msg,
           Wn_upd, Wm_upd, Wg_upd, Wlin_upd, W_final):
    f32 = jnp.float32
    row = edge_index[0]
    col = edge_index[1]

    xp = jnp.zeros((NP_, D_IN), f32).at[:N].set(x)
    pos16 = jnp.zeros((NP_, 16), f32).at[:N, :3].set(pos)
    rowp = jnp.concatenate(
        [row, jnp.full((EP - E,), N, jnp.int32)]).reshape(EP // CHUNK, CHUNK)
    colp = jnp.concatenate(
        [col, jnp.zeros((EP - E,), jnp.int32)]).reshape(EP // CHUNK, CHUNK)
    batch2 = jnp.concatenate(
        [batch, jnp.full((NP_ - N,), NG, jnp.int32)]).reshape(NP_, 1)
    zeros_acc = jnp.zeros((NP_, 128), f32)

    eye = jnp.eye(S0, dtype=f32)
    wn_msg_p = jnp.zeros((L, H, HP), f32).at[:, :, :H].set(Wn_msg)
    we_p = jnp.zeros((L, 16, HP), f32).at[:, :4, :H].set(We_msg)
    wg_msg_ext = jnp.zeros((L, S0, HP), f32)
    wg_msg_ext = wg_msg_ext.at[:, :, :S0].set(eye)
    wg_msg_ext = wg_msg_ext.at[:, :, S0:H].set(Wg_msg)
    wg_upd_ext = jnp.concatenate(
        [jnp.broadcast_to(eye, (L, S0, S0)), Wg_upd], axis=2)

    pr = _sc_gather16(pos16, rowp)
    pc = _sc_gather16(pos16, colp)
    ea = _edge_attr_call(pr, pc)

    h, g = _prologue_call(xp, W_in, wn_msg_p[0])
    for l in range(L):
        gath = _sc_gather256(g, colp)
        gated = _gate_call(gath, ea, we_p[l], wg_msg_ext[l])
        agg = _sc_scatter(gated, rowp, zeros_acc)
        wn_next = wn_msg_p[l + 1] if l + 1 < L else None
        res = _update_call(h, agg, Wlin_msg[l], Wn_upd[l], Wm_upd[l],
                           wg_upd_ext[l], Wlin_upd[l], wn_next)
        if wn_next is not None:
            h, g = res
        else:
            (h,) = res

    return _pool_call(h, batch2, W_final)
